# Initial kernel scaffold; baseline (speedup 1.0000x reference)
import jax
import jax.numpy as jnp
from jax.experimental import pallas as pl


def kernel(x, edge_index, w1, root1, b1, w2, root2, b2):
    raise NotImplementedError("write your pallas kernel here")


# speedup vs baseline: 13.1555x; 13.1555x over previous
import functools

import jax
import jax.numpy as jnp
from jax import lax
from jax.experimental import pallas as pl
from jax.experimental.pallas import tpu as pltpu
from jax.experimental.pallas import tpu_sc as plsc

N = 10000
E = 320000
F_IN = 128
HID = 16
NCLS = 10

NC = 2
NS = 16
NW = NC * NS
CHUNK = 128
CPT = 79
EPAD = NW * CPT * CHUNK
NACC = 10240
RPT = NACC // NS
RBLK = 1000
GRID = N // RBLK


def _tc_a_body(x_ref, wa_ref, t_ref, xr_ref):
    y = jnp.dot(x_ref[...], wa_ref[...], preferred_element_type=jnp.float32)
    ones = jnp.ones((RBLK, 1), jnp.float32)
    zeros = jnp.zeros((RBLK, HID - 1), jnp.float32)
    t_ref[...] = jnp.concatenate([y[:, :HID], ones, zeros], axis=1)
    xr_ref[...] = y[:, HID:]


def _tc_a(x, wa):
    return pl.pallas_call(
        _tc_a_body,
        grid=(GRID,),
        in_specs=[
            pl.BlockSpec((RBLK, F_IN), lambda i: (i, 0)),
            pl.BlockSpec((F_IN, 2 * HID), lambda i: (0, 0)),
        ],
        out_specs=[
            pl.BlockSpec((RBLK, 2 * HID), lambda i: (i, 0)),
            pl.BlockSpec((RBLK, HID), lambda i: (i, 0)),
        ],
        out_shape=[
            jax.ShapeDtypeStruct((N, 2 * HID), jnp.float32),
            jax.ShapeDtypeStruct((N, HID), jnp.float32),
        ],
    )(x, wa)


def _make_sc_scatter(width):
    @functools.partial(
        pl.kernel,
        out_type=jax.ShapeDtypeStruct((NC, NACC, width), jnp.float32),
        mesh=plsc.VectorSubcoreMesh(core_axis_name="c", subcore_axis_name="s"),
        compiler_params=pltpu.CompilerParams(use_tc_tiling_on_sc=False),
        scratch_types=[
            pltpu.VMEM((CPT, CHUNK), jnp.int32),
            pltpu.VMEM((CPT, CHUNK), jnp.int32),
            pltpu.VMEM((CHUNK, width), jnp.float32),
            pltpu.VMEM((RPT, width), jnp.float32),
            pltpu.VMEM_SHARED((NACC, width), jnp.float32),
            pltpu.SemaphoreType.DMA,
        ],
    )
    def sc_scatter(src_hbm, dst_hbm, table_hbm, out_hbm,
                   src_v, dst_v, rows_v, zb_v, accum, sem):
        cid = lax.axis_index("c")
        sid = lax.axis_index("s")
        wid = cid * NS + sid

        zeros16 = jnp.zeros((16,), jnp.float32)

        def zrow(i, carry):
            for j in range(width // 16):
                zb_v[i, pl.ds(j * 16, 16)] = zeros16
            return carry

        lax.fori_loop(0, RPT, zrow, 0)
        pltpu.sync_copy(zb_v, accum.at[pl.ds(sid * RPT, RPT)])

        pltpu.sync_copy(src_hbm.at[wid], src_v)
        pltpu.sync_copy(dst_hbm.at[wid], dst_v)
        plsc.subcore_barrier()

        def chunk(c, carry):
            pltpu.async_copy(table_hbm.at[src_v.at[c]], rows_v, sem).wait()
            pltpu.sync_copy(rows_v, accum.at[dst_v.at[c]], add=True)
            return carry

        lax.fori_loop(0, CPT, chunk, 0)
        plsc.subcore_barrier()

        pltpu.sync_copy(accum.at[pl.ds(sid * RPT, RPT)],
                        out_hbm.at[cid, pl.ds(sid * RPT, RPT)])

    return sc_scatter


_sc_scatter_32 = _make_sc_scatter(2 * HID)
_sc_scatter_16 = _make_sc_scatter(HID)


def _tc_b_body(p_ref, xr_ref, wb_ref, b1_ref, b2_ref, t2_ref, misc_ref):
    s = p_ref[0] + p_ref[1]
    cnt = s[:, HID:HID + 1]
    inv = 1.0 / jnp.maximum(cnt, 1.0)
    pre = s[:, :HID] * inv + xr_ref[...] + b1_ref[0][None, :]
    h = jnp.where(pre > 0, pre,
                  jnp.exp(jnp.where(pre > 0, 0.0, pre)) - 1.0)
    y2 = jnp.dot(h, wb_ref[...], preferred_element_type=jnp.float32)
    zeros6 = jnp.zeros((RBLK, HID - NCLS), jnp.float32)
    t2_ref[...] = jnp.concatenate([y2[:, :NCLS], zeros6], axis=1)
    hr2 = y2[:, NCLS:] + b2_ref[0][None, :NCLS]
    zeros5 = jnp.zeros((RBLK, HID - NCLS - 1), jnp.float32)
    misc_ref[...] = jnp.concatenate([hr2, inv, zeros5], axis=1)


def _tc_b(partials1, xr1, wb, b1, b2):
    return pl.pallas_call(
        _tc_b_body,
        grid=(GRID,),
        in_specs=[
            pl.BlockSpec((NC, RBLK, 2 * HID), lambda i: (0, i, 0)),
            pl.BlockSpec((RBLK, HID), lambda i: (i, 0)),
            pl.BlockSpec((HID, 2 * NCLS), lambda i: (0, 0)),
            pl.BlockSpec((1, HID), lambda i: (0, 0)),
            pl.BlockSpec((1, HID), lambda i: (0, 0)),
        ],
        out_specs=[
            pl.BlockSpec((RBLK, HID), lambda i: (i, 0)),
            pl.BlockSpec((RBLK, HID), lambda i: (i, 0)),
        ],
        out_shape=[
            jax.ShapeDtypeStruct((N, HID), jnp.float32),
            jax.ShapeDtypeStruct((N, HID), jnp.float32),
        ],
    )(partials1, xr1, wb, b1, b2)


def _tc_c_body(p_ref, misc_ref, out_ref):
    s = p_ref[0] + p_ref[1]
    inv = misc_ref[:, NCLS:NCLS + 1]
    o = s[:, :NCLS] * inv + misc_ref[:, :NCLS]
    m = jnp.max(o, axis=1, keepdims=True)
    z = o - m
    ls = z - jnp.log(jnp.sum(jnp.exp(z), axis=1, keepdims=True))
    pad = jnp.zeros((RBLK, HID - NCLS), jnp.float32)
    out_ref[...] = jnp.concatenate([ls, pad], axis=1)


def _tc_c(partials2, misc2):
    return pl.pallas_call(
        _tc_c_body,
        grid=(GRID,),
        in_specs=[
            pl.BlockSpec((NC, RBLK, HID), lambda i: (0, i, 0)),
            pl.BlockSpec((RBLK, HID), lambda i: (i, 0)),
        ],
        out_specs=pl.BlockSpec((RBLK, HID), lambda i: (i, 0)),
        out_shape=jax.ShapeDtypeStruct((N, HID), jnp.float32),
    )(partials2, misc2)


def kernel(x, edge_index, w1, root1, b1, w2, root2, b2):
    wa = jnp.concatenate([w1[1], root1], axis=1)
    wb = jnp.concatenate([w2[1], root2], axis=1)
    b1r = b1.reshape(1, HID)
    b2r = jnp.zeros((1, HID), jnp.float32).at[0, :NCLS].set(b2)

    pad = EPAD - E
    src = jnp.concatenate([edge_index[0], jnp.zeros((pad,), jnp.int32)])
    dst = jnp.concatenate([edge_index[1],
                           jnp.full((pad,), N, jnp.int32)])
    src3 = src.reshape(NW, CPT, CHUNK)
    dst3 = dst.reshape(NW, CPT, CHUNK)

    table1, xr1 = _tc_a(x, wa)
    partials1 = _sc_scatter_32(src3, dst3, table1)
    table2, misc2 = _tc_b(partials1, xr1, wb, b1r, b2r)
    partials2 = _sc_scatter_16(src3, dst3, table2)
    res = _tc_c(partials2, misc2)
    return res[:, :NCLS]


# speedup vs baseline: 16.9841x; 1.2910x over previous
import functools

import jax
import jax.numpy as jnp
from jax import lax
from jax.experimental import pallas as pl
from jax.experimental.pallas import tpu as pltpu
from jax.experimental.pallas import tpu_sc as plsc

N = 10000
E = 320000
F_IN = 128
HID = 16
NCLS = 10

NC = 2
NS = 16
NW = NC * NS
CHUNK = 128
CPT = 80
EPAD = NW * CPT * CHUNK
NACC = 10240
RPT = NACC // NS
RBLK = 1000
GRID = N // RBLK
BBLK = 1024
BGRID = NACC // BBLK


def _tc_a_body(x_ref, wa_ref, t_ref, xr_ref):
    y = jnp.dot(x_ref[...], wa_ref[...], preferred_element_type=jnp.float32)
    t_ref[...] = y[:, :HID]
    xr_ref[...] = y[:, HID:]


def _tc_a(x, wa):
    return pl.pallas_call(
        _tc_a_body,
        grid=(GRID,),
        in_specs=[
            pl.BlockSpec((RBLK, F_IN), lambda i: (i, 0)),
            pl.BlockSpec((F_IN, 2 * HID), lambda i: (0, 0)),
        ],
        out_specs=[
            pl.BlockSpec((RBLK, HID), lambda i: (i, 0)),
            pl.BlockSpec((RBLK, HID), lambda i: (i, 0)),
        ],
        out_shape=[
            jax.ShapeDtypeStruct((NACC, HID), jnp.float32),
            jax.ShapeDtypeStruct((NACC, HID), jnp.float32),
        ],
    )(x, wa)


def _make_sc_scatter(with_counts):
    width = HID
    if with_counts:
        out_type = [
            jax.ShapeDtypeStruct((NC, NACC, width), jnp.float32),
            jax.ShapeDtypeStruct((NW, NACC), jnp.float32),
        ]
    else:
        out_type = jax.ShapeDtypeStruct((NC, NACC, width), jnp.float32)

    @functools.partial(
        pl.kernel,
        out_type=out_type,
        mesh=plsc.VectorSubcoreMesh(core_axis_name="c", subcore_axis_name="s"),
        compiler_params=pltpu.CompilerParams(use_tc_tiling_on_sc=False,
                                             needs_layout_passes=False),
        scratch_types=[
            pltpu.VMEM((CPT, CHUNK), jnp.int32),
            pltpu.VMEM((CPT, CHUNK), jnp.int32),
            pltpu.VMEM((CHUNK, width), jnp.float32),
            pltpu.VMEM((CHUNK, width), jnp.float32),
            pltpu.VMEM((RPT, width), jnp.float32),
            pltpu.VMEM((NACC,), jnp.float32),
            pltpu.VMEM_SHARED((NACC, width), jnp.float32),
            pltpu.SemaphoreType.DMA,
            pltpu.SemaphoreType.DMA,
        ],
    )
    def sc_scatter(src_hbm, dst_hbm, table_hbm, *rest):
        if with_counts:
            out_hbm, cnt_hbm = rest[0], rest[1]
            scr = rest[2:]
        else:
            out_hbm = rest[0]
            scr = rest[1:]
        src_v, dst_v, rb0, rb1, zb_v, cnt_v, accum, gs0, gs1 = scr

        cid = lax.axis_index("c")
        sid = lax.axis_index("s")
        wid = cid * NS + sid

        zeros16 = jnp.zeros((16,), jnp.float32)
        ones16 = jnp.ones((16,), jnp.float32)

        def zrow(i, carry):
            zb_v[i, pl.ds(0, 16)] = zeros16
            cnt_v[pl.ds(i * 16, 16)] = zeros16
            return carry

        lax.fori_loop(0, RPT, zrow, 0)
        pltpu.sync_copy(zb_v, accum.at[pl.ds(sid * RPT, RPT)])

        pltpu.sync_copy(src_hbm.at[wid], src_v)
        pltpu.sync_copy(dst_hbm.at[wid], dst_v)
        plsc.subcore_barrier()

        def fire(c, rb, sem):
            pltpu.async_copy(table_hbm.at[src_v.at[c]], rb, sem)

        def drain(rb, sem):
            pltpu.make_async_copy(table_hbm.at[src_v.at[0]], rb, sem).wait()

        def count(c):
            if with_counts:
                for j in range(CHUNK // 16):
                    idx = dst_v[c, pl.ds(j * 16, 16)]
                    plsc.addupdate_scatter(cnt_v, [idx], ones16)

        fire(0, rb0, gs0)
        fire(1, rb1, gs1)

        def chunk_pair(i, carry):
            c0 = 2 * i
            drain(rb0, gs0)
            pltpu.sync_copy(rb0, accum.at[dst_v.at[c0]], add=True)

            @pl.when(i < CPT // 2 - 1)
            def _():
                fire(c0 + 2, rb0, gs0)

            count(c0)
            drain(rb1, gs1)
            pltpu.sync_copy(rb1, accum.at[dst_v.at[c0 + 1]], add=True)

            @pl.when(i < CPT // 2 - 1)
            def _():
                fire(c0 + 3, rb1, gs1)

            count(c0 + 1)
            return carry

        lax.fori_loop(0, CPT // 2, chunk_pair, 0)
        plsc.subcore_barrier()

        pltpu.sync_copy(accum.at[pl.ds(sid * RPT, RPT)],
                        out_hbm.at[cid, pl.ds(sid * RPT, RPT)])
        if with_counts:
            pltpu.sync_copy(cnt_v, cnt_hbm.at[wid])

    return sc_scatter


_sc_scatter_l1 = _make_sc_scatter(True)
_sc_scatter_l2 = _make_sc_scatter(False)


def _tc_b_body(p_ref, c_ref, xr_ref, wb_ref, b1_ref, b2_ref, t2_ref, misc_ref):
    s = p_ref[0] + p_ref[1]
    i = pl.program_id(0)
    cs = c_ref[:, pl.ds(i * BBLK, BBLK)]
    cnt = jnp.sum(cs, axis=0)[:, None]
    inv = 1.0 / jnp.maximum(cnt, 1.0)
    pre = s * inv + xr_ref[...] + b1_ref[0][None, :]
    h = jnp.where(pre > 0, pre,
                  jnp.exp(jnp.where(pre > 0, 0.0, pre)) - 1.0)
    y2 = jnp.dot(h, wb_ref[...], preferred_element_type=jnp.float32)
    zeros6 = jnp.zeros((BBLK, HID - NCLS), jnp.float32)
    t2_ref[...] = jnp.concatenate([y2[:, :NCLS], zeros6], axis=1)
    hr2 = y2[:, NCLS:] + b2_ref[0][None, :NCLS]
    zeros5 = jnp.zeros((BBLK, HID - NCLS - 1), jnp.float32)
    misc_ref[...] = jnp.concatenate([hr2, inv, zeros5], axis=1)


def _tc_b(partials1, cnts, xr1, wb, b1, b2):
    return pl.pallas_call(
        _tc_b_body,
        grid=(BGRID,),
        in_specs=[
            pl.BlockSpec((NC, BBLK, HID), lambda i: (0, i, 0)),
            pl.BlockSpec((NW, NACC), lambda i: (0, 0)),
            pl.BlockSpec((BBLK, HID), lambda i: (i, 0)),
            pl.BlockSpec((HID, 2 * NCLS), lambda i: (0, 0)),
            pl.BlockSpec((1, HID), lambda i: (0, 0)),
            pl.BlockSpec((1, HID), lambda i: (0, 0)),
        ],
        out_specs=[
            pl.BlockSpec((BBLK, HID), lambda i: (i, 0)),
            pl.BlockSpec((BBLK, HID), lambda i: (i, 0)),
        ],
        out_shape=[
            jax.ShapeDtypeStruct((NACC, HID), jnp.float32),
            jax.ShapeDtypeStruct((NACC, HID), jnp.float32),
        ],
    )(partials1, cnts, xr1, wb, b1, b2)


def _tc_c_body(p_ref, misc_ref, out_ref):
    s = p_ref[0] + p_ref[1]
    inv = misc_ref[:, NCLS:NCLS + 1]
    o = s[:, :NCLS] * inv + misc_ref[:, :NCLS]
    m = jnp.max(o, axis=1, keepdims=True)
    z = o - m
    ls = z - jnp.log(jnp.sum(jnp.exp(z), axis=1, keepdims=True))
    pad = jnp.zeros((BBLK, HID - NCLS), jnp.float32)
    out_ref[...] = jnp.concatenate([ls, pad], axis=1)


def _tc_c(partials2, misc2):
    return pl.pallas_call(
        _tc_c_body,
        grid=(BGRID,),
        in_specs=[
            pl.BlockSpec((NC, BBLK, HID), lambda i: (0, i, 0)),
            pl.BlockSpec((BBLK, HID), lambda i: (i, 0)),
        ],
        out_specs=pl.BlockSpec((BBLK, HID), lambda i: (i, 0)),
        out_shape=jax.ShapeDtypeStruct((NACC, HID), jnp.float32),
    )(partials2, misc2)


def kernel(x, edge_index, w1, root1, b1, w2, root2, b2):
    wa = jnp.concatenate([w1[1], root1], axis=1)
    wb = jnp.concatenate([w2[1], root2], axis=1)
    b1r = b1.reshape(1, HID)
    b2r = jnp.zeros((1, HID), jnp.float32).at[0, :NCLS].set(b2)

    pad = EPAD - E
    src = jnp.concatenate([edge_index[0], jnp.zeros((pad,), jnp.int32)])
    dst = jnp.concatenate([edge_index[1],
                           jnp.full((pad,), N, jnp.int32)])
    src3 = src.reshape(NW, CPT, CHUNK)
    dst3 = dst.reshape(NW, CPT, CHUNK)

    table1, xr1 = _tc_a(x, wa)
    partials1, cnts = _sc_scatter_l1(src3, dst3, table1)
    table2, misc2 = _tc_b(partials1, cnts, xr1, wb, b1r, b2r)
    partials2 = _sc_scatter_l2(src3, dst3, table2)
    res = _tc_c(partials2, misc2)
    return res[:N, :NCLS]


# speedup vs baseline: 17.5062x; 1.0307x over previous
import functools

import jax
import jax.numpy as jnp
from jax import lax
from jax.experimental import pallas as pl
from jax.experimental.pallas import tpu as pltpu
from jax.experimental.pallas import tpu_sc as plsc

N = 10000
E = 320000
F_IN = 128
HID = 16
NCLS = 10

NC = 2
NS = 16
NW = NC * NS
CHUNK = 128
CPT = 80
EPAD = NW * CPT * CHUNK
NACC = 10240
RPT = NACC // NS
RBLK = 1000
GRID = N // RBLK
BBLK = 1024
BGRID = NACC // BBLK


def _tc_a_body(x_ref, wa_ref, t_ref, xr_ref):
    y = jnp.dot(x_ref[...], wa_ref[...], preferred_element_type=jnp.float32)
    t_ref[...] = y[:, :HID]
    xr_ref[...] = y[:, HID:]


def _tc_a(x, wa):
    return pl.pallas_call(
        _tc_a_body,
        grid=(GRID,),
        in_specs=[
            pl.BlockSpec((RBLK, F_IN), lambda i: (i, 0)),
            pl.BlockSpec((F_IN, 2 * HID), lambda i: (0, 0)),
        ],
        out_specs=[
            pl.BlockSpec((RBLK, HID), lambda i: (i, 0)),
            pl.BlockSpec((RBLK, HID), lambda i: (i, 0)),
        ],
        out_shape=[
            jax.ShapeDtypeStruct((NACC, HID), jnp.float32),
            jax.ShapeDtypeStruct((NACC, HID), jnp.float32),
        ],
    )(x, wa)


def _make_sc_scatter(with_counts):
    width = HID
    if with_counts:
        out_type = [
            jax.ShapeDtypeStruct((NC, NACC, width), jnp.float32),
            jax.ShapeDtypeStruct((NW, NACC), jnp.float32),
        ]
    else:
        out_type = jax.ShapeDtypeStruct((NC, NACC, width), jnp.float32)

    @functools.partial(
        pl.kernel,
        out_type=out_type,
        mesh=plsc.VectorSubcoreMesh(core_axis_name="c", subcore_axis_name="s"),
        compiler_params=pltpu.CompilerParams(use_tc_tiling_on_sc=False,
                                             needs_layout_passes=False),
        scratch_types=[
            pltpu.VMEM((CPT, CHUNK), jnp.int32),
            pltpu.VMEM((CPT, CHUNK), jnp.int32),
            [pltpu.VMEM((CHUNK, width), jnp.float32) for _ in range(8)],
            pltpu.VMEM((RPT, width), jnp.float32),
            pltpu.VMEM((NACC,), jnp.float32),
            pltpu.VMEM_SHARED((NACC, width), jnp.float32),
            pltpu.SemaphoreType.DMA,
            pltpu.SemaphoreType.DMA,
            pltpu.SemaphoreType.DMA,
            pltpu.SemaphoreType.DMA,
        ],
    )
    def sc_scatter(src_hbm, dst_hbm, table_hbm, *rest):
        if with_counts:
            out_hbm, cnt_hbm = rest[0], rest[1]
            scr = rest[2:]
        else:
            out_hbm = rest[0]
            scr = rest[1:]
        src_v, dst_v, rbufs, zb_v, cnt_v, accum, gsA, gsB, ssA, ssB = scr
        bufA, bufB = rbufs[:4], rbufs[4:]

        cid = lax.axis_index("c")
        sid = lax.axis_index("s")
        wid = cid * NS + sid

        zeros16 = jnp.zeros((16,), jnp.float32)
        ones16 = jnp.ones((16,), jnp.float32)

        def zrow(i, carry):
            zb_v[i, pl.ds(0, 16)] = zeros16
            cnt_v[pl.ds(i * 16, 16)] = zeros16
            return carry

        lax.fori_loop(0, RPT, zrow, 0)
        pltpu.sync_copy(zb_v, accum.at[pl.ds(sid * RPT, RPT)])

        pltpu.sync_copy(src_hbm.at[wid], src_v)
        pltpu.sync_copy(dst_hbm.at[wid], dst_v)
        plsc.subcore_barrier()

        def fire_gathers(c0, bufs, sem):
            for k in range(4):
                pltpu.async_copy(table_hbm.at[src_v.at[c0 + k]], bufs[k], sem)

        def drain_gathers(bufs, sem):
            for k in range(4):
                pltpu.make_async_copy(
                    table_hbm.at[src_v.at[0]], bufs[k], sem).wait()

        def fire_scatters(c0, bufs, sem):
            for k in range(4):
                pltpu.async_copy(bufs[k], accum.at[dst_v.at[c0 + k]], sem,
                                 add=True)

        def drain_scatters(bufs, sem):
            for k in range(4):
                pltpu.make_async_copy(
                    bufs[k], accum.at[dst_v.at[0]], sem).wait()

        def count(c0):
            if with_counts:
                for k in range(4):
                    for j in range(CHUNK // 16):
                        idx = dst_v[c0 + k, pl.ds(j * 16, 16)]
                        plsc.addupdate_scatter(cnt_v, [idx], ones16)

        NG = CPT // 8

        fire_gathers(0, bufA, gsA)

        def group_pair(i, carry):
            c0 = 8 * i

            @pl.when(i > 0)
            def _():
                drain_scatters(bufB, ssB)

            fire_gathers(c0 + 4, bufB, gsB)
            drain_gathers(bufA, gsA)
            fire_scatters(c0, bufA, ssA)
            count(c0)

            drain_scatters(bufA, ssA)

            @pl.when(i < NG - 1)
            def _():
                fire_gathers(c0 + 8, bufA, gsA)

            drain_gathers(bufB, gsB)
            fire_scatters(c0 + 4, bufB, ssB)
            count(c0 + 4)
            return carry

        lax.fori_loop(0, NG, group_pair, 0)
        drain_scatters(bufB, ssB)
        plsc.subcore_barrier()

        pltpu.sync_copy(accum.at[pl.ds(sid * RPT, RPT)],
                        out_hbm.at[cid, pl.ds(sid * RPT, RPT)])
        if with_counts:
            pltpu.sync_copy(cnt_v, cnt_hbm.at[wid])

    return sc_scatter


_sc_scatter_l1 = _make_sc_scatter(True)
_sc_scatter_l2 = _make_sc_scatter(False)


def _tc_b_body(p_ref, c_ref, xr_ref, wb_ref, b1_ref, b2_ref, t2_ref, misc_ref):
    s = p_ref[0] + p_ref[1]
    i = pl.program_id(0)
    cs = c_ref[:, pl.ds(i * BBLK, BBLK)]
    cnt = jnp.sum(cs, axis=0)[:, None]
    inv = 1.0 / jnp.maximum(cnt, 1.0)
    pre = s * inv + xr_ref[...] + b1_ref[0][None, :]
    h = jnp.where(pre > 0, pre,
                  jnp.exp(jnp.where(pre > 0, 0.0, pre)) - 1.0)
    y2 = jnp.dot(h, wb_ref[...], preferred_element_type=jnp.float32)
    zeros6 = jnp.zeros((BBLK, HID - NCLS), jnp.float32)
    t2_ref[...] = jnp.concatenate([y2[:, :NCLS], zeros6], axis=1)
    hr2 = y2[:, NCLS:] + b2_ref[0][None, :NCLS]
    zeros5 = jnp.zeros((BBLK, HID - NCLS - 1), jnp.float32)
    misc_ref[...] = jnp.concatenate([hr2, inv, zeros5], axis=1)


def _tc_b(partials1, cnts, xr1, wb, b1, b2):
    return pl.pallas_call(
        _tc_b_body,
        grid=(BGRID,),
        in_specs=[
            pl.BlockSpec((NC, BBLK, HID), lambda i: (0, i, 0)),
            pl.BlockSpec((NW, NACC), lambda i: (0, 0)),
            pl.BlockSpec((BBLK, HID), lambda i: (i, 0)),
            pl.BlockSpec((HID, 2 * NCLS), lambda i: (0, 0)),
            pl.BlockSpec((1, HID), lambda i: (0, 0)),
            pl.BlockSpec((1, HID), lambda i: (0, 0)),
        ],
        out_specs=[
            pl.BlockSpec((BBLK, HID), lambda i: (i, 0)),
            pl.BlockSpec((BBLK, HID), lambda i: (i, 0)),
        ],
        out_shape=[
            jax.ShapeDtypeStruct((NACC, HID), jnp.float32),
            jax.ShapeDtypeStruct((NACC, HID), jnp.float32),
        ],
    )(partials1, cnts, xr1, wb, b1, b2)


def _tc_c_body(p_ref, misc_ref, out_ref):
    s = p_ref[0] + p_ref[1]
    inv = misc_ref[:, NCLS:NCLS + 1]
    o = s[:, :NCLS] * inv + misc_ref[:, :NCLS]
    m = jnp.max(o, axis=1, keepdims=True)
    z = o - m
    ls = z - jnp.log(jnp.sum(jnp.exp(z), axis=1, keepdims=True))
    pad = jnp.zeros((BBLK, HID - NCLS), jnp.float32)
    out_ref[...] = jnp.concatenate([ls, pad], axis=1)


def _tc_c(partials2, misc2):
    return pl.pallas_call(
        _tc_c_body,
        grid=(BGRID,),
        in_specs=[
            pl.BlockSpec((NC, BBLK, HID), lambda i: (0, i, 0)),
            pl.BlockSpec((BBLK, HID), lambda i: (i, 0)),
        ],
        out_specs=pl.BlockSpec((BBLK, HID), lambda i: (i, 0)),
        out_shape=jax.ShapeDtypeStruct((NACC, HID), jnp.float32),
    )(partials2, misc2)


def kernel(x, edge_index, w1, root1, b1, w2, root2, b2):
    wa = jnp.concatenate([w1[1], root1], axis=1)
    wb = jnp.concatenate([w2[1], root2], axis=1)
    b1r = b1.reshape(1, HID)
    b2r = jnp.zeros((1, HID), jnp.float32).at[0, :NCLS].set(b2)

    pad = EPAD - E
    src = jnp.concatenate([edge_index[0], jnp.zeros((pad,), jnp.int32)])
    dst = jnp.concatenate([edge_index[1],
                           jnp.full((pad,), N, jnp.int32)])
    src3 = src.reshape(NW, CPT, CHUNK)
    dst3 = dst.reshape(NW, CPT, CHUNK)

    table1, xr1 = _tc_a(x, wa)
    partials1, cnts = _sc_scatter_l1(src3, dst3, table1)
    table2, misc2 = _tc_b(partials1, cnts, xr1, wb, b1r, b2r)
    partials2 = _sc_scatter_l2(src3, dst3, table2)
    res = _tc_c(partials2, misc2)
    return res[:N, :NCLS]


# speedup vs baseline: 19.5823x; 1.1186x over previous
import functools

import jax
import jax.numpy as jnp
from jax import lax
from jax.experimental import pallas as pl
from jax.experimental.pallas import tpu as pltpu
from jax.experimental.pallas import tpu_sc as plsc

N = 10000
E = 320000
F_IN = 128
HID = 16
NCLS = 10

NC = 2
NS = 16
NW = NC * NS
CHUNK = 128
CPT = 80
EPAD = NW * CPT * CHUNK
NACC = 10240
RPT = NACC // NS
NPK = N * HID // 128
APK = NACC * HID // 128


def _tc_a_body(x8_ref, w_ref, t_ref, xr_ref):
    y = jnp.dot(x8_ref[...], w_ref[...], preferred_element_type=jnp.float32)
    t_ref[...] = y[:, :128]
    xr_ref[...] = y[:, 128:]


def _tc_a(x8, w8a):
    return pl.pallas_call(
        _tc_a_body,
        grid=(1,),
        in_specs=[
            pl.BlockSpec((NPK, 8 * F_IN), lambda i: (0, 0)),
            pl.BlockSpec((8 * F_IN, 256), lambda i: (0, 0)),
        ],
        out_specs=[
            pl.BlockSpec((NPK, 128), lambda i: (0, 0)),
            pl.BlockSpec((NPK, 128), lambda i: (0, 0)),
        ],
        out_shape=[
            jax.ShapeDtypeStruct((NPK, 128), jnp.float32),
            jax.ShapeDtypeStruct((NPK, 128), jnp.float32),
        ],
    )(x8, w8a)


def _make_sc_scatter(with_counts):
    width = HID
    if with_counts:
        out_type = [
            jax.ShapeDtypeStruct((NC, NACC, width), jnp.float32),
            jax.ShapeDtypeStruct((NC, NACC, width), jnp.float32),
        ]
    else:
        out_type = jax.ShapeDtypeStruct((NC, NACC, width), jnp.float32)

    scratch = [
        pltpu.VMEM((CPT, CHUNK), jnp.int32),
        pltpu.VMEM((CPT, CHUNK), jnp.int32),
        [pltpu.VMEM((CHUNK, width), jnp.float32) for _ in range(8)],
        pltpu.VMEM((RPT, width), jnp.float32),
        pltpu.VMEM((CHUNK, width), jnp.float32),
        pltpu.VMEM_SHARED((NACC, width), jnp.float32),
        pltpu.VMEM_SHARED((NACC, width), jnp.float32),
        pltpu.SemaphoreType.DMA,
        pltpu.SemaphoreType.DMA,
        pltpu.SemaphoreType.DMA,
        pltpu.SemaphoreType.DMA,
    ]

    @functools.partial(
        pl.kernel,
        out_type=out_type,
        mesh=plsc.VectorSubcoreMesh(core_axis_name="c", subcore_axis_name="s"),
        compiler_params=pltpu.CompilerParams(use_tc_tiling_on_sc=False,
                                             needs_layout_passes=False),
        scratch_types=scratch,
    )
    def sc_scatter(src_hbm, dst_hbm, table_hbm, *rest):
        if with_counts:
            out_hbm, cnt_hbm = rest[0], rest[1]
            scr = rest[2:]
        else:
            out_hbm = rest[0]
            scr = rest[1:]
        (src_v, dst_v, rbufs, zb_v, ones_v, accum, accum_c,
         gsA, gsB, ssA, ssB) = scr
        bufA, bufB = rbufs[:4], rbufs[4:]

        cid = lax.axis_index("c")
        sid = lax.axis_index("s")
        wid = cid * NS + sid

        icp1 = pltpu.async_copy(src_hbm.at[wid], src_v, gsA)
        icp2 = pltpu.async_copy(dst_hbm.at[wid], dst_v, gsB)

        zeros16 = jnp.zeros((16,), jnp.float32)
        ones16 = jnp.ones((16,), jnp.float32)

        def zrow(i, carry):
            zb_v[i, pl.ds(0, 16)] = zeros16
            return carry

        lax.fori_loop(0, RPT, zrow, 0)
        if with_counts:
            def orow(i, carry):
                ones_v[i, pl.ds(0, 16)] = ones16
                return carry

            lax.fori_loop(0, CHUNK, orow, 0)
        pltpu.sync_copy(zb_v, accum.at[pl.ds(sid * RPT, RPT)])
        if with_counts:
            pltpu.sync_copy(zb_v, accum_c.at[pl.ds(sid * RPT, RPT)])
        icp1.wait()
        icp2.wait()
        plsc.subcore_barrier()

        def fire_gathers(c0, bufs, sem):
            for k in range(4):
                pltpu.async_copy(table_hbm.at[src_v.at[c0 + k]], bufs[k], sem)

        def drain_gathers(bufs, sem):
            for k in range(4):
                pltpu.make_async_copy(
                    table_hbm.at[src_v.at[0]], bufs[k], sem).wait()

        def fire_scatters(c0, bufs, sem):
            for k in range(4):
                pltpu.async_copy(bufs[k], accum.at[dst_v.at[c0 + k]], sem,
                                 add=True)
                if with_counts:
                    pltpu.async_copy(ones_v, accum_c.at[dst_v.at[c0 + k]],
                                     sem, add=True)

        def drain_scatters(bufs, sem):
            for k in range(4):
                pltpu.make_async_copy(
                    bufs[k], accum.at[dst_v.at[0]], sem).wait()
                if with_counts:
                    pltpu.make_async_copy(
                        ones_v, accum_c.at[dst_v.at[0]], sem).wait()

        NG = CPT // 8

        fire_gathers(0, bufA, gsA)

        def group_pair(i, carry):
            c0 = 8 * i

            @pl.when(i > 0)
            def _():
                drain_scatters(bufB, ssB)

            fire_gathers(c0 + 4, bufB, gsB)
            drain_gathers(bufA, gsA)
            fire_scatters(c0, bufA, ssA)

            drain_scatters(bufA, ssA)

            @pl.when(i < NG - 1)
            def _():
                fire_gathers(c0 + 8, bufA, gsA)

            drain_gathers(bufB, gsB)
            fire_scatters(c0 + 4, bufB, ssB)
            return carry

        lax.fori_loop(0, NG, group_pair, 0)
        drain_scatters(bufB, ssB)
        plsc.subcore_barrier()

        pltpu.sync_copy(accum.at[pl.ds(sid * RPT, RPT)],
                        out_hbm.at[cid, pl.ds(sid * RPT, RPT)])
        if with_counts:
            pltpu.sync_copy(accum_c.at[pl.ds(sid * RPT, RPT)],
                            cnt_hbm.at[cid, pl.ds(sid * RPT, RPT)])

    return sc_scatter


_sc_scatter_l1 = _make_sc_scatter(True)
_sc_scatter_l2 = _make_sc_scatter(False)


def _tc_b_body(pv_ref, pc_ref, xr_ref, wt2_ref, wm_ref, b1_ref, b2_ref,
               m10_ref, t2_ref, misc_ref):
    sp = pv_ref[0] + pv_ref[1]
    cntp = pc_ref[0] + pc_ref[1]
    invp = 1.0 / jnp.maximum(cntp, 1.0)
    pre = sp * invp + xr_ref[...] + b1_ref[...]
    h = jnp.where(pre > 0, pre,
                  jnp.exp(jnp.where(pre > 0, 0.0, pre)) - 1.0)
    t2_ref[...] = jnp.dot(h, wt2_ref[...], preferred_element_type=jnp.float32)
    hr2 = jnp.dot(h, wm_ref[...], preferred_element_type=jnp.float32)
    misc_ref[...] = hr2 + b2_ref[...] + invp * m10_ref[...]


def _tc_b(p1v, p1c, xr1p, w8t2, w8m, b1t, b2t, m10):
    vec = pl.BlockSpec((1, 128), lambda i: (0, 0))
    mat = pl.BlockSpec((128, 128), lambda i: (0, 0))
    pk = pl.BlockSpec((NPK, 128), lambda i: (0, 0))
    return pl.pallas_call(
        _tc_b_body,
        grid=(1,),
        in_specs=[
            pl.BlockSpec((NC, NPK, 128), lambda i: (0, 0, 0)),
            pl.BlockSpec((NC, NPK, 128), lambda i: (0, 0, 0)),
            pk, mat, mat, vec, vec, vec,
        ],
        out_specs=[pk, pk],
        out_shape=[
            jax.ShapeDtypeStruct((NPK, 128), jnp.float32),
            jax.ShapeDtypeStruct((NPK, 128), jnp.float32),
        ],
    )(p1v, p1c, xr1p, w8t2, w8m, b1t, b2t, m10)


def _tc_c_body(p_ref, misc_ref, out_ref):
    sp = p_ref[0] + p_ref[1]
    misc = misc_ref[...]

    lane = lax.broadcasted_iota(jnp.int32, (128, 128), 0)
    col = lax.broadcasted_iota(jnp.int32, (128, 128), 1)
    grp_eq = (lane // 16) == (col // 16)

    bmat = grp_eq & ((lane % 16) == 10) & ((col % 16) < NCLS)
    invb = jnp.dot(misc, bmat.astype(jnp.float32),
                   preferred_element_type=jnp.float32)

    lane1 = lax.broadcasted_iota(jnp.int32, (NPK, 128), 1)
    valid = (lane1 % 16) < NCLS
    o = jnp.where(valid, sp * invb + misc, -1e30)

    m = o
    for k in (1, 2, 4, 8):
        rot = grp_eq & ((col % 16) == ((lane + k) % 16))
        m = jnp.maximum(m, jnp.dot(m, rot.astype(jnp.float32),
                                   preferred_element_type=jnp.float32))
    z = o - m
    e = jnp.exp(z)
    ssum = jnp.dot(e, grp_eq.astype(jnp.float32),
                   preferred_element_type=jnp.float32)
    out_ref[...] = z - jnp.log(ssum)


def _tc_c(p2v, miscp):
    pk = pl.BlockSpec((NPK, 128), lambda i: (0, 0))
    return pl.pallas_call(
        _tc_c_body,
        grid=(1,),
        in_specs=[
            pl.BlockSpec((NC, NPK, 128), lambda i: (0, 0, 0)),
            pk,
        ],
        out_specs=pk,
        out_shape=jax.ShapeDtypeStruct((NPK, 128), jnp.float32),
    )(p2v, miscp)


def kernel(x, edge_index, w1, root1, b1, w2, root2, b2):
    eye8 = jnp.eye(8, dtype=jnp.float32)
    x8 = x.reshape(NPK, 8 * F_IN)
    w8a = jnp.concatenate(
        [jnp.kron(eye8, w1[1]), jnp.kron(eye8, root1)], axis=1)
    wt2p = jnp.pad(w2[1], ((0, 0), (0, HID - NCLS)))
    wmp = jnp.pad(root2, ((0, 0), (0, HID - NCLS)))
    w8t2 = jnp.kron(eye8, wt2p)
    w8m = jnp.kron(eye8, wmp)
    b1t = jnp.tile(b1, 8)[None, :]
    b2t = jnp.tile(jnp.pad(b2, (0, HID - NCLS)), 8)[None, :]
    m10 = (jnp.arange(128) % 16 == 10).astype(jnp.float32)[None, :]

    pad = EPAD - E
    src = jnp.concatenate([edge_index[0], jnp.zeros((pad,), jnp.int32)])
    dst = jnp.concatenate([edge_index[1],
                           jnp.full((pad,), N, jnp.int32)])
    src3 = src.reshape(NW, CPT, CHUNK)
    dst3 = dst.reshape(NW, CPT, CHUNK)

    t1p, xr1p = _tc_a(x8, w8a)
    pv1, pc1 = _sc_scatter_l1(src3, dst3, t1p.reshape(N, HID))
    p1v = pv1.reshape(NC, APK, 128)[:, :NPK]
    p1c = pc1.reshape(NC, APK, 128)[:, :NPK]
    t2p, miscp = _tc_b(p1v, p1c, xr1p, w8t2, w8m, b1t, b2t, m10)
    pv2 = _sc_scatter_l2(src3, dst3, t2p.reshape(N, HID))
    p2v = pv2.reshape(NC, APK, 128)[:, :NPK]
    res = _tc_c(p2v, miscp)
    return res.reshape(N, HID)[:, :NCLS]


# speedup vs baseline: 20.6482x; 1.0544x over previous
import functools

import jax
import jax.numpy as jnp
from jax import lax
from jax.experimental import pallas as pl
from jax.experimental.pallas import tpu as pltpu
from jax.experimental.pallas import tpu_sc as plsc

N = 10000
E = 320000
F_IN = 128
HID = 16
NCLS = 10

NC = 2
NS = 16
NW = NC * NS
CHUNK = 128
CPT = 80
EPAD = NW * CPT * CHUNK
NACC = 10240
RPT = NACC // NS
NPK = N * HID // 128
APK = NACC * HID // 128


def _tc_a_body(x8_ref, w_ref, t_ref, xr_ref):
    y = jnp.dot(x8_ref[...], w_ref[...], preferred_element_type=jnp.float32)
    t_ref[...] = y[:, :128]
    xr_ref[...] = y[:, 128:]


def _tc_a(x8, w8a):
    return pl.pallas_call(
        _tc_a_body,
        grid=(1,),
        in_specs=[
            pl.BlockSpec((NPK, 8 * F_IN), lambda i: (0, 0)),
            pl.BlockSpec((8 * F_IN, 256), lambda i: (0, 0)),
        ],
        out_specs=[
            pl.BlockSpec((NPK, 128), lambda i: (0, 0)),
            pl.BlockSpec((NPK, 128), lambda i: (0, 0)),
        ],
        out_shape=[
            jax.ShapeDtypeStruct((NPK, 128), jnp.float32),
            jax.ShapeDtypeStruct((NPK, 128), jnp.float32),
        ],
    )(x8, w8a)


def _make_sc_scatter(with_counts):
    width = HID
    if with_counts:
        out_type = [
            jax.ShapeDtypeStruct((NC, NACC, width), jnp.float32),
            jax.ShapeDtypeStruct((NC, NACC, width), jnp.float32),
        ]
    else:
        out_type = jax.ShapeDtypeStruct((NC, NACC, width), jnp.float32)

    scratch = [
        pltpu.VMEM((CPT, CHUNK), jnp.int32),
        pltpu.VMEM((CPT, CHUNK), jnp.int32),
        [pltpu.VMEM((CHUNK, width), jnp.float32) for _ in range(16)],
        pltpu.VMEM((RPT, width), jnp.float32),
        pltpu.VMEM((CHUNK, width), jnp.float32),
        pltpu.VMEM_SHARED((NACC, width), jnp.float32),
        pltpu.VMEM_SHARED((NACC, width), jnp.float32),
        pltpu.SemaphoreType.DMA,
        pltpu.SemaphoreType.DMA,
        pltpu.SemaphoreType.DMA,
        pltpu.SemaphoreType.DMA,
    ]

    @functools.partial(
        pl.kernel,
        out_type=out_type,
        mesh=plsc.VectorSubcoreMesh(core_axis_name="c", subcore_axis_name="s"),
        compiler_params=pltpu.CompilerParams(use_tc_tiling_on_sc=False,
                                             needs_layout_passes=False),
        scratch_types=scratch,
    )
    def sc_scatter(src_hbm, dst_hbm, table_hbm, *rest):
        if with_counts:
            out_hbm, cnt_hbm = rest[0], rest[1]
            scr = rest[2:]
        else:
            out_hbm = rest[0]
            scr = rest[1:]
        (src_v, dst_v, rbufs, zb_v, ones_v, accum, accum_c,
         gsA, gsB, ssA, ssB) = scr
        bufA, bufB = rbufs[:8], rbufs[8:]

        cid = lax.axis_index("c")
        sid = lax.axis_index("s")
        wid = cid * NS + sid

        icp1 = pltpu.async_copy(src_hbm.at[wid], src_v, gsA)
        icp2 = pltpu.async_copy(dst_hbm.at[wid], dst_v, gsB)

        zeros16 = jnp.zeros((16,), jnp.float32)
        ones16 = jnp.ones((16,), jnp.float32)

        def zrow(i, carry):
            for u in range(8):
                zb_v[8 * i + u, pl.ds(0, 16)] = zeros16
            return carry

        lax.fori_loop(0, RPT // 8, zrow, 0)
        if with_counts:
            def orow(i, carry):
                for u in range(8):
                    ones_v[8 * i + u, pl.ds(0, 16)] = ones16
                return carry

            lax.fori_loop(0, CHUNK // 8, orow, 0)
        pltpu.sync_copy(zb_v, accum.at[pl.ds(sid * RPT, RPT)])
        if with_counts:
            pltpu.sync_copy(zb_v, accum_c.at[pl.ds(sid * RPT, RPT)])
        icp1.wait()
        icp2.wait()
        plsc.subcore_barrier()

        def fire_gathers(c0, bufs, sem):
            for k in range(8):
                pltpu.async_copy(table_hbm.at[src_v.at[c0 + k]], bufs[k], sem)

        def drain_gathers(bufs, sem):
            for k in range(8):
                pltpu.make_async_copy(
                    table_hbm.at[src_v.at[0]], bufs[k], sem).wait()

        def fire_scatters(c0, bufs, sem):
            for k in range(8):
                pltpu.async_copy(bufs[k], accum.at[dst_v.at[c0 + k]], sem,
                                 add=True)
                if with_counts:
                    pltpu.async_copy(ones_v, accum_c.at[dst_v.at[c0 + k]],
                                     sem, add=True)

        def drain_scatters(bufs, sem):
            for k in range(8):
                pltpu.make_async_copy(
                    bufs[k], accum.at[dst_v.at[0]], sem).wait()
                if with_counts:
                    pltpu.make_async_copy(
                        ones_v, accum_c.at[dst_v.at[0]], sem).wait()

        NG = CPT // 16

        fire_gathers(0, bufA, gsA)

        def group_pair(i, carry):
            c0 = 16 * i

            @pl.when(i > 0)
            def _():
                drain_scatters(bufB, ssB)

            fire_gathers(c0 + 8, bufB, gsB)
            drain_gathers(bufA, gsA)
            fire_scatters(c0, bufA, ssA)

            drain_scatters(bufA, ssA)

            @pl.when(i < NG - 1)
            def _():
                fire_gathers(c0 + 16, bufA, gsA)

            drain_gathers(bufB, gsB)
            fire_scatters(c0 + 8, bufB, ssB)
            return carry

        lax.fori_loop(0, NG, group_pair, 0)
        drain_scatters(bufB, ssB)
        plsc.subcore_barrier()

        fcp1 = pltpu.async_copy(accum.at[pl.ds(sid * RPT, RPT)],
                                out_hbm.at[cid, pl.ds(sid * RPT, RPT)], gsA)
        if with_counts:
            pltpu.async_copy(accum_c.at[pl.ds(sid * RPT, RPT)],
                             cnt_hbm.at[cid, pl.ds(sid * RPT, RPT)],
                             gsB).wait()
        fcp1.wait()

    return sc_scatter


_sc_scatter_l1 = _make_sc_scatter(True)
_sc_scatter_l2 = _make_sc_scatter(False)


def _tc_b_body(pv_ref, pc_ref, xr_ref, wt2_ref, wm_ref, b1_ref, b2_ref,
               m10_ref, t2_ref, misc_ref):
    sp = pv_ref[0] + pv_ref[1]
    cntp = pc_ref[0] + pc_ref[1]
    invp = 1.0 / jnp.maximum(cntp, 1.0)
    pre = sp * invp + xr_ref[...] + b1_ref[...]
    h = jnp.where(pre > 0, pre,
                  jnp.exp(jnp.where(pre > 0, 0.0, pre)) - 1.0)
    t2_ref[...] = jnp.dot(h, wt2_ref[...], preferred_element_type=jnp.float32)
    hr2 = jnp.dot(h, wm_ref[...], preferred_element_type=jnp.float32)
    misc_ref[...] = hr2 + b2_ref[...] + invp * m10_ref[...]


def _tc_b(p1v, p1c, xr1p, w8t2, w8m, b1t, b2t, m10):
    vec = pl.BlockSpec((1, 128), lambda i: (0, 0))
    mat = pl.BlockSpec((128, 128), lambda i: (0, 0))
    pk = pl.BlockSpec((NPK, 128), lambda i: (0, 0))
    return pl.pallas_call(
        _tc_b_body,
        grid=(1,),
        in_specs=[
            pl.BlockSpec((NC, NPK, 128), lambda i: (0, 0, 0)),
            pl.BlockSpec((NC, NPK, 128), lambda i: (0, 0, 0)),
            pk, mat, mat, vec, vec, vec,
        ],
        out_specs=[pk, pk],
        out_shape=[
            jax.ShapeDtypeStruct((NPK, 128), jnp.float32),
            jax.ShapeDtypeStruct((NPK, 128), jnp.float32),
        ],
    )(p1v, p1c, xr1p, w8t2, w8m, b1t, b2t, m10)


def _tc_c_body(p_ref, misc_ref, out_ref):
    sp = p_ref[0] + p_ref[1]
    misc = misc_ref[...]

    lane = lax.broadcasted_iota(jnp.int32, (128, 128), 0)
    col = lax.broadcasted_iota(jnp.int32, (128, 128), 1)
    grp_eq = (lane // 16) == (col // 16)

    bmat = grp_eq & ((lane % 16) == 10) & ((col % 16) < NCLS)
    invb = jnp.dot(misc, bmat.astype(jnp.float32),
                   preferred_element_type=jnp.float32)

    lane1 = lax.broadcasted_iota(jnp.int32, (NPK, 128), 1)
    valid = (lane1 % 16) < NCLS
    o = jnp.where(valid, sp * invb + misc, -1e30)

    m = o
    for k in (1, 2, 4, 8):
        rot = grp_eq & ((col % 16) == ((lane + k) % 16))
        m = jnp.maximum(m, jnp.dot(m, rot.astype(jnp.float32),
                                   preferred_element_type=jnp.float32))
    z = o - m
    e = jnp.exp(z)
    ssum = jnp.dot(e, grp_eq.astype(jnp.float32),
                   preferred_element_type=jnp.float32)
    out_ref[...] = z - jnp.log(ssum)


def _tc_c(p2v, miscp):
    pk = pl.BlockSpec((NPK, 128), lambda i: (0, 0))
    return pl.pallas_call(
        _tc_c_body,
        grid=(1,),
        in_specs=[
            pl.BlockSpec((NC, NPK, 128), lambda i: (0, 0, 0)),
            pk,
        ],
        out_specs=pk,
        out_shape=jax.ShapeDtypeStruct((NPK, 128), jnp.float32),
    )(p2v, miscp)


def kernel(x, edge_index, w1, root1, b1, w2, root2, b2):
    eye8 = jnp.eye(8, dtype=jnp.float32)
    x8 = x.reshape(NPK, 8 * F_IN)
    w8a = jnp.concatenate(
        [jnp.kron(eye8, w1[1]), jnp.kron(eye8, root1)], axis=1)
    wt2p = jnp.pad(w2[1], ((0, 0), (0, HID - NCLS)))
    wmp = jnp.pad(root2, ((0, 0), (0, HID - NCLS)))
    w8t2 = jnp.kron(eye8, wt2p)
    w8m = jnp.kron(eye8, wmp)
    b1t = jnp.tile(b1, 8)[None, :]
    b2t = jnp.tile(jnp.pad(b2, (0, HID - NCLS)), 8)[None, :]
    m10 = (jnp.arange(128) % 16 == 10).astype(jnp.float32)[None, :]

    pad = EPAD - E
    src = jnp.concatenate([edge_index[0], jnp.zeros((pad,), jnp.int32)])
    dst = jnp.concatenate([edge_index[1],
                           jnp.full((pad,), N, jnp.int32)])
    src3 = src.reshape(NW, CPT, CHUNK)
    dst3 = dst.reshape(NW, CPT, CHUNK)

    t1p, xr1p = _tc_a(x8, w8a)
    pv1, pc1 = _sc_scatter_l1(src3, dst3, t1p.reshape(N, HID))
    p1v = pv1.reshape(NC, APK, 128)[:, :NPK]
    p1c = pc1.reshape(NC, APK, 128)[:, :NPK]
    t2p, miscp = _tc_b(p1v, p1c, xr1p, w8t2, w8m, b1t, b2t, m10)
    pv2 = _sc_scatter_l2(src3, dst3, t2p.reshape(N, HID))
    p2v = pv2.reshape(NC, APK, 128)[:, :NPK]
    res = _tc_c(p2v, miscp)
    return res.reshape(N, HID)[:, :NCLS]


# speedup vs baseline: 30.3415x; 1.4694x over previous
import functools

import jax
import jax.numpy as jnp
from jax import lax
from jax.experimental import pallas as pl
from jax.experimental.pallas import tpu as pltpu
from jax.experimental.pallas import tpu_sc as plsc

N = 10000
E = 320000
F_IN = 128
HID = 16
NCLS = 10

NC = 2
NS = 16
NW = NC * NS
CHUNK = 128
CPT = 80
EPAD = NW * CPT * CHUNK
NACC = 10240
RPT = NACC // NS
NPK = N * HID // 128
APK = NACC * HID // 128


def _tc_a_body(x8_ref, w_ref, t_ref, xr_ref):
    y = jnp.dot(x8_ref[...], w_ref[...], preferred_element_type=jnp.float32)
    t_ref[...] = y[:, :128]
    xr_ref[...] = y[:, 128:]


def _tc_a(x8, w8a):
    return pl.pallas_call(
        _tc_a_body,
        grid=(1,),
        in_specs=[
            pl.BlockSpec((NPK, 8 * F_IN), lambda i: (0, 0)),
            pl.BlockSpec((8 * F_IN, 256), lambda i: (0, 0)),
        ],
        out_specs=[
            pl.BlockSpec((NPK, 128), lambda i: (0, 0)),
            pl.BlockSpec((NPK, 128), lambda i: (0, 0)),
        ],
        out_shape=[
            jax.ShapeDtypeStruct((NPK, 128), jnp.float32),
            jax.ShapeDtypeStruct((NPK, 128), jnp.float32),
        ],
    )(x8, w8a)


def _make_sc_scatter(with_counts):
    width = HID
    if with_counts:
        out_type = [
            jax.ShapeDtypeStruct((NC, NACC, width), jnp.float32),
            jax.ShapeDtypeStruct((NC, NACC, width), jnp.float32),
        ]
    else:
        out_type = jax.ShapeDtypeStruct((NC, NACC, width), jnp.float32)

    scratch = [
        pltpu.VMEM((CPT, CHUNK), jnp.int32),
        pltpu.VMEM((CPT, CHUNK), jnp.int32),
        [pltpu.VMEM((CHUNK, width), jnp.float32) for _ in range(16)],
        pltpu.VMEM((RPT, width), jnp.float32),
        pltpu.VMEM((CHUNK, width), jnp.float32),
        pltpu.VMEM_SHARED((N, width), jnp.float32),
        pltpu.VMEM_SHARED((NACC, width), jnp.float32),
        pltpu.VMEM_SHARED((NACC, width), jnp.float32),
        pltpu.SemaphoreType.DMA,
        pltpu.SemaphoreType.DMA,
        pltpu.SemaphoreType.DMA,
        pltpu.SemaphoreType.DMA,
    ]

    @functools.partial(
        pl.kernel,
        out_type=out_type,
        mesh=plsc.VectorSubcoreMesh(core_axis_name="c", subcore_axis_name="s"),
        compiler_params=pltpu.CompilerParams(use_tc_tiling_on_sc=False,
                                             needs_layout_passes=False),
        scratch_types=scratch,
    )
    def sc_scatter(src_hbm, dst_hbm, table_hbm, *rest):
        if with_counts:
            out_hbm, cnt_hbm = rest[0], rest[1]
            scr = rest[2:]
        else:
            out_hbm = rest[0]
            scr = rest[1:]
        (src_v, dst_v, rbufs, zb_v, ones_v, table_spm, accum, accum_c,
         gsA, gsB, ssA, ssB) = scr
        bufA, bufB = rbufs[:8], rbufs[8:]

        cid = lax.axis_index("c")
        sid = lax.axis_index("s")
        wid = cid * NS + sid

        TS = N // NS
        icp1 = pltpu.async_copy(src_hbm.at[wid], src_v, gsA)
        icp2 = pltpu.async_copy(dst_hbm.at[wid], dst_v, gsB)
        icp3 = pltpu.async_copy(table_hbm.at[pl.ds(sid * TS, TS)],
                                table_spm.at[pl.ds(sid * TS, TS)], ssA)

        zeros16 = jnp.zeros((16,), jnp.float32)
        ones16 = jnp.ones((16,), jnp.float32)

        def zrow(i, carry):
            for u in range(8):
                zb_v[8 * i + u, pl.ds(0, 16)] = zeros16
            return carry

        lax.fori_loop(0, RPT // 8, zrow, 0)
        if with_counts:
            def orow(i, carry):
                for u in range(8):
                    ones_v[8 * i + u, pl.ds(0, 16)] = ones16
                return carry

            lax.fori_loop(0, CHUNK // 8, orow, 0)
        pltpu.sync_copy(zb_v, accum.at[pl.ds(sid * RPT, RPT)])
        if with_counts:
            pltpu.sync_copy(zb_v, accum_c.at[pl.ds(sid * RPT, RPT)])
        icp1.wait()
        icp2.wait()
        icp3.wait()
        plsc.subcore_barrier()

        def fire_gathers(c0, bufs, sem):
            for k in range(8):
                pltpu.async_copy(table_spm.at[src_v.at[c0 + k]], bufs[k], sem)

        def drain_gathers(bufs, sem):
            for k in range(8):
                pltpu.make_async_copy(
                    table_spm.at[src_v.at[0]], bufs[k], sem).wait()

        def fire_scatters(c0, bufs, sem):
            for k in range(8):
                pltpu.async_copy(bufs[k], accum.at[dst_v.at[c0 + k]], sem,
                                 add=True)
                if with_counts:
                    pltpu.async_copy(ones_v, accum_c.at[dst_v.at[c0 + k]],
                                     sem, add=True)

        def drain_scatters(bufs, sem):
            for k in range(8):
                pltpu.make_async_copy(
                    bufs[k], accum.at[dst_v.at[0]], sem).wait()
                if with_counts:
                    pltpu.make_async_copy(
                        ones_v, accum_c.at[dst_v.at[0]], sem).wait()

        NG = CPT // 16

        fire_gathers(0, bufA, gsA)

        def group_pair(i, carry):
            c0 = 16 * i

            @pl.when(i > 0)
            def _():
                drain_scatters(bufB, ssB)

            fire_gathers(c0 + 8, bufB, gsB)
            drain_gathers(bufA, gsA)
            fire_scatters(c0, bufA, ssA)

            drain_scatters(bufA, ssA)

            @pl.when(i < NG - 1)
            def _():
                fire_gathers(c0 + 16, bufA, gsA)

            drain_gathers(bufB, gsB)
            fire_scatters(c0 + 8, bufB, ssB)
            return carry

        lax.fori_loop(0, NG, group_pair, 0)
        drain_scatters(bufB, ssB)
        plsc.subcore_barrier()

        fcp1 = pltpu.async_copy(accum.at[pl.ds(sid * RPT, RPT)],
                                out_hbm.at[cid, pl.ds(sid * RPT, RPT)], gsA)
        if with_counts:
            pltpu.async_copy(accum_c.at[pl.ds(sid * RPT, RPT)],
                             cnt_hbm.at[cid, pl.ds(sid * RPT, RPT)],
                             gsB).wait()
        fcp1.wait()

    return sc_scatter


_sc_scatter_l1 = _make_sc_scatter(True)
_sc_scatter_l2 = _make_sc_scatter(False)


def _tc_b_body(pv_ref, pc_ref, xr_ref, wt2_ref, wm_ref, b1_ref, b2_ref,
               m10_ref, t2_ref, misc_ref):
    sp = pv_ref[0] + pv_ref[1]
    cntp = pc_ref[0] + pc_ref[1]
    invp = 1.0 / jnp.maximum(cntp, 1.0)
    pre = sp * invp + xr_ref[...] + b1_ref[...]
    h = jnp.where(pre > 0, pre,
                  jnp.exp(jnp.where(pre > 0, 0.0, pre)) - 1.0)
    t2_ref[...] = jnp.dot(h, wt2_ref[...], preferred_element_type=jnp.float32)
    hr2 = jnp.dot(h, wm_ref[...], preferred_element_type=jnp.float32)
    misc_ref[...] = hr2 + b2_ref[...] + invp * m10_ref[...]


def _tc_b(p1v, p1c, xr1p, w8t2, w8m, b1t, b2t, m10):
    vec = pl.BlockSpec((1, 128), lambda i: (0, 0))
    mat = pl.BlockSpec((128, 128), lambda i: (0, 0))
    pk = pl.BlockSpec((NPK, 128), lambda i: (0, 0))
    return pl.pallas_call(
        _tc_b_body,
        grid=(1,),
        in_specs=[
            pl.BlockSpec((NC, NPK, 128), lambda i: (0, 0, 0)),
            pl.BlockSpec((NC, NPK, 128), lambda i: (0, 0, 0)),
            pk, mat, mat, vec, vec, vec,
        ],
        out_specs=[pk, pk],
        out_shape=[
            jax.ShapeDtypeStruct((NPK, 128), jnp.float32),
            jax.ShapeDtypeStruct((NPK, 128), jnp.float32),
        ],
    )(p1v, p1c, xr1p, w8t2, w8m, b1t, b2t, m10)


def _tc_c_body(p_ref, misc_ref, out_ref):
    sp = p_ref[0] + p_ref[1]
    misc = misc_ref[...]

    lane = lax.broadcasted_iota(jnp.int32, (128, 128), 0)
    col = lax.broadcasted_iota(jnp.int32, (128, 128), 1)
    grp_eq = (lane // 16) == (col // 16)

    bmat = grp_eq & ((lane % 16) == 10) & ((col % 16) < NCLS)
    invb = jnp.dot(misc, bmat.astype(jnp.float32),
                   preferred_element_type=jnp.float32)

    lane1 = lax.broadcasted_iota(jnp.int32, (NPK, 128), 1)
    valid = (lane1 % 16) < NCLS
    o = jnp.where(valid, sp * invb + misc, -1e30)

    m = o
    for k in (1, 2, 4, 8):
        rot = grp_eq & ((col % 16) == ((lane + k) % 16))
        m = jnp.maximum(m, jnp.dot(m, rot.astype(jnp.float32),
                                   preferred_element_type=jnp.float32))
    z = o - m
    e = jnp.exp(z)
    ssum = jnp.dot(e, grp_eq.astype(jnp.float32),
                   preferred_element_type=jnp.float32)
    out_ref[...] = z - jnp.log(ssum)


def _tc_c(p2v, miscp):
    pk = pl.BlockSpec((NPK, 128), lambda i: (0, 0))
    return pl.pallas_call(
        _tc_c_body,
        grid=(1,),
        in_specs=[
            pl.BlockSpec((NC, NPK, 128), lambda i: (0, 0, 0)),
            pk,
        ],
        out_specs=pk,
        out_shape=jax.ShapeDtypeStruct((NPK, 128), jnp.float32),
    )(p2v, miscp)


def kernel(x, edge_index, w1, root1, b1, w2, root2, b2):
    eye8 = jnp.eye(8, dtype=jnp.float32)
    x8 = x.reshape(NPK, 8 * F_IN)
    w8a = jnp.concatenate(
        [jnp.kron(eye8, w1[1]), jnp.kron(eye8, root1)], axis=1)
    wt2p = jnp.pad(w2[1], ((0, 0), (0, HID - NCLS)))
    wmp = jnp.pad(root2, ((0, 0), (0, HID - NCLS)))
    w8t2 = jnp.kron(eye8, wt2p)
    w8m = jnp.kron(eye8, wmp)
    b1t = jnp.tile(b1, 8)[None, :]
    b2t = jnp.tile(jnp.pad(b2, (0, HID - NCLS)), 8)[None, :]
    m10 = (jnp.arange(128) % 16 == 10).astype(jnp.float32)[None, :]

    pad = EPAD - E
    src = jnp.concatenate([edge_index[0], jnp.zeros((pad,), jnp.int32)])
    dst = jnp.concatenate([edge_index[1],
                           jnp.full((pad,), N, jnp.int32)])
    src3 = src.reshape(NW, CPT, CHUNK)
    dst3 = dst.reshape(NW, CPT, CHUNK)

    t1p, xr1p = _tc_a(x8, w8a)
    pv1, pc1 = _sc_scatter_l1(src3, dst3, t1p.reshape(N, HID))
    p1v = pv1.reshape(NC, APK, 128)[:, :NPK]
    p1c = pc1.reshape(NC, APK, 128)[:, :NPK]
    t2p, miscp = _tc_b(p1v, p1c, xr1p, w8t2, w8m, b1t, b2t, m10)
    pv2 = _sc_scatter_l2(src3, dst3, t2p.reshape(N, HID))
    p2v = pv2.reshape(NC, APK, 128)[:, :NPK]
    res = _tc_c(p2v, miscp)
    return res.reshape(N, HID)[:, :NCLS]


# speedup vs baseline: 31.7687x; 1.0470x over previous
import functools

import jax
import jax.numpy as jnp
from jax import lax
from jax.experimental import pallas as pl
from jax.experimental.pallas import tpu as pltpu
from jax.experimental.pallas import tpu_sc as plsc

N = 10000
E = 320000
F_IN = 128
HID = 16
NCLS = 10

NC = 2
NS = 16
NW = NC * NS
CHUNK = 128
CPT = 80
EPAD = NW * CPT * CHUNK
NACC = 10240
RPT = NACC // NS
NPK = N * HID // 128
APK = NACC * HID // 128


def _tc_a_body(x8_ref, w_ref, t_ref, xr_ref):
    y = jnp.dot(x8_ref[...], w_ref[...], preferred_element_type=jnp.float32)
    t_ref[...] = y[:, :128]
    xr_ref[...] = y[:, 128:]


def _tc_a(x8, w8a):
    return pl.pallas_call(
        _tc_a_body,
        grid=(1,),
        in_specs=[
            pl.BlockSpec((NPK, 8 * F_IN), lambda i: (0, 0)),
            pl.BlockSpec((8 * F_IN, 256), lambda i: (0, 0)),
        ],
        out_specs=[
            pl.BlockSpec((NPK, 128), lambda i: (0, 0)),
            pl.BlockSpec((NPK, 128), lambda i: (0, 0)),
        ],
        out_shape=[
            jax.ShapeDtypeStruct((NPK, 128), jnp.float32),
            jax.ShapeDtypeStruct((NPK, 128), jnp.float32),
        ],
    )(x8, w8a)


def _make_sc_scatter(with_counts):
    width = HID
    if with_counts:
        out_type = [
            jax.ShapeDtypeStruct((NC, NACC, width), jnp.float32),
            jax.ShapeDtypeStruct((NC, NACC, width), jnp.float32),
        ]
    else:
        out_type = jax.ShapeDtypeStruct((NC, NACC, width), jnp.float32)

    scratch = [
        pltpu.VMEM((CPT, CHUNK), jnp.int32),
        pltpu.VMEM((CPT, CHUNK), jnp.int32),
        [pltpu.VMEM((CHUNK, width), jnp.float32) for _ in range(16)],
        pltpu.VMEM((RPT, width), jnp.float32),
        pltpu.VMEM((CHUNK, width), jnp.float32),
        pltpu.VMEM_SHARED((N, width), jnp.float32),
        pltpu.VMEM_SHARED((NACC, width), jnp.float32),
        pltpu.VMEM_SHARED((NACC, width), jnp.float32),
        pltpu.SemaphoreType.DMA,
        pltpu.SemaphoreType.DMA,
        pltpu.SemaphoreType.DMA,
        pltpu.SemaphoreType.DMA,
    ]

    @functools.partial(
        pl.kernel,
        out_type=out_type,
        mesh=plsc.VectorSubcoreMesh(core_axis_name="c", subcore_axis_name="s"),
        compiler_params=pltpu.CompilerParams(use_tc_tiling_on_sc=False,
                                             needs_layout_passes=False),
        scratch_types=scratch,
    )
    def sc_scatter(src_hbm, dst_hbm, table_hbm, *rest):
        if with_counts:
            out_hbm, cnt_hbm = rest[0], rest[1]
            scr = rest[2:]
        else:
            out_hbm = rest[0]
            scr = rest[1:]
        (src_v, dst_v, rbufs, zb_v, ones_v, table_spm, accum, accum_c,
         gsA, gsB, ssA, ssB) = scr
        bufA, bufB = rbufs[:8], rbufs[8:]

        cid = lax.axis_index("c")
        sid = lax.axis_index("s")
        wid = cid * NS + sid

        TS = N // NS
        icp1 = pltpu.async_copy(src_hbm.at[wid], src_v, gsA)
        icp2 = pltpu.async_copy(dst_hbm.at[wid], dst_v, gsB)
        icp3 = pltpu.async_copy(table_hbm.at[pl.ds(sid * TS, TS)],
                                table_spm.at[pl.ds(sid * TS, TS)], ssA)

        zeros16 = jnp.zeros((16,), jnp.float32)
        ones16 = jnp.ones((16,), jnp.float32)

        def zrow(i, carry):
            for u in range(8):
                zb_v[8 * i + u, pl.ds(0, 16)] = zeros16
            return carry

        lax.fori_loop(0, RPT // 8, zrow, 0)
        if with_counts:
            def orow(i, carry):
                for u in range(8):
                    ones_v[8 * i + u, pl.ds(0, 16)] = ones16
                return carry

            lax.fori_loop(0, CHUNK // 8, orow, 0)
        pltpu.sync_copy(zb_v, accum.at[pl.ds(sid * RPT, RPT)])
        if with_counts:
            pltpu.sync_copy(zb_v, accum_c.at[pl.ds(sid * RPT, RPT)])
        icp1.wait()
        icp2.wait()
        icp3.wait()
        plsc.subcore_barrier()

        def fire_gathers(c0, bufs, sem):
            for k in range(8):
                pltpu.async_copy(table_spm.at[src_v.at[c0 + k]], bufs[k], sem)

        def drain_gathers(bufs, sem):
            for k in range(8):
                pltpu.make_async_copy(
                    table_spm.at[src_v.at[0]], bufs[k], sem).wait()

        def fire_scatters(c0, bufs, sem):
            for k in range(8):
                pltpu.async_copy(bufs[k], accum.at[dst_v.at[c0 + k]], sem,
                                 add=True)
                if with_counts:
                    pltpu.async_copy(ones_v, accum_c.at[dst_v.at[c0 + k]],
                                     sem, add=True)

        def drain_scatters(bufs, sem):
            for k in range(8):
                pltpu.make_async_copy(
                    bufs[k], accum.at[dst_v.at[0]], sem).wait()
                if with_counts:
                    pltpu.make_async_copy(
                        ones_v, accum_c.at[dst_v.at[0]], sem).wait()

        NG = CPT // 16

        fire_gathers(0, bufA, gsA)

        def group_pair(i, carry):
            c0 = 16 * i

            @pl.when(i > 0)
            def _():
                drain_scatters(bufB, ssB)

            fire_gathers(c0 + 8, bufB, gsB)
            drain_gathers(bufA, gsA)
            fire_scatters(c0, bufA, ssA)

            drain_scatters(bufA, ssA)

            @pl.when(i < NG - 1)
            def _():
                fire_gathers(c0 + 16, bufA, gsA)

            drain_gathers(bufB, gsB)
            fire_scatters(c0 + 8, bufB, ssB)
            return carry

        lax.fori_loop(0, NG, group_pair, 0)
        drain_scatters(bufB, ssB)
        plsc.subcore_barrier()

        fcp1 = pltpu.async_copy(accum.at[pl.ds(sid * RPT, RPT)],
                                out_hbm.at[cid, pl.ds(sid * RPT, RPT)], gsA)
        if with_counts:
            pltpu.async_copy(accum_c.at[pl.ds(sid * RPT, RPT)],
                             cnt_hbm.at[cid, pl.ds(sid * RPT, RPT)],
                             gsB).wait()
        fcp1.wait()

    return sc_scatter


_sc_scatter_l1 = _make_sc_scatter(True)
_sc_scatter_l2 = _make_sc_scatter(False)


def _tc_b_body(pv_ref, pc_ref, xr_ref, wt2_ref, wm_ref, b1_ref, b2_ref,
               m10_ref, t2_ref, misc_ref):
    sp = pv_ref[0, :NPK] + pv_ref[1, :NPK]
    cntp = pc_ref[0, :NPK] + pc_ref[1, :NPK]
    invp = 1.0 / jnp.maximum(cntp, 1.0)
    pre = sp * invp + xr_ref[...] + b1_ref[...]
    h = jnp.where(pre > 0, pre,
                  jnp.exp(jnp.where(pre > 0, 0.0, pre)) - 1.0)
    t2_ref[...] = jnp.dot(h, wt2_ref[...], preferred_element_type=jnp.float32)
    hr2 = jnp.dot(h, wm_ref[...], preferred_element_type=jnp.float32)
    misc_ref[...] = hr2 + b2_ref[...] + invp * m10_ref[...]


def _tc_b(p1v, p1c, xr1p, w8t2, w8m, b1t, b2t, m10):
    vec = pl.BlockSpec((1, 128), lambda i: (0, 0))
    mat = pl.BlockSpec((128, 128), lambda i: (0, 0))
    pk = pl.BlockSpec((NPK, 128), lambda i: (0, 0))
    return pl.pallas_call(
        _tc_b_body,
        grid=(1,),
        in_specs=[
            pl.BlockSpec((NC, APK, 128), lambda i: (0, 0, 0)),
            pl.BlockSpec((NC, APK, 128), lambda i: (0, 0, 0)),
            pk, mat, mat, vec, vec, vec,
        ],
        out_specs=[pk, pk],
        out_shape=[
            jax.ShapeDtypeStruct((NPK, 128), jnp.float32),
            jax.ShapeDtypeStruct((NPK, 128), jnp.float32),
        ],
    )(p1v, p1c, xr1p, w8t2, w8m, b1t, b2t, m10)


def _tc_c_body(p_ref, misc_ref, out_ref):
    sp = p_ref[0, :NPK] + p_ref[1, :NPK]
    misc = misc_ref[...]

    lane = lax.broadcasted_iota(jnp.int32, (128, 128), 0)
    col = lax.broadcasted_iota(jnp.int32, (128, 128), 1)
    grp_eq = (lane // 16) == (col // 16)

    bmat = grp_eq & ((lane % 16) == 10) & ((col % 16) < NCLS)
    invb = jnp.dot(misc, bmat.astype(jnp.float32),
                   preferred_element_type=jnp.float32)

    lane1 = lax.broadcasted_iota(jnp.int32, (NPK, 128), 1)
    valid = (lane1 % 16) < NCLS
    o = jnp.where(valid, sp * invb + misc, -1e30)

    m = o
    for k in (1, 2, 4, 8):
        rot = grp_eq & ((col % 16) == ((lane + k) % 16))
        m = jnp.maximum(m, jnp.dot(m, rot.astype(jnp.float32),
                                   preferred_element_type=jnp.float32))
    z = o - m
    e = jnp.exp(z)
    ssum = jnp.dot(e, grp_eq.astype(jnp.float32),
                   preferred_element_type=jnp.float32)
    out_ref[...] = z - jnp.log(ssum)


def _tc_c(p2v, miscp):
    pk = pl.BlockSpec((NPK, 128), lambda i: (0, 0))
    return pl.pallas_call(
        _tc_c_body,
        grid=(1,),
        in_specs=[
            pl.BlockSpec((NC, APK, 128), lambda i: (0, 0, 0)),
            pk,
        ],
        out_specs=pk,
        out_shape=jax.ShapeDtypeStruct((NPK, 128), jnp.float32),
    )(p2v, miscp)


def kernel(x, edge_index, w1, root1, b1, w2, root2, b2):
    eye8 = jnp.eye(8, dtype=jnp.float32)
    x8 = x.reshape(NPK, 8 * F_IN)
    w8a = jnp.concatenate(
        [jnp.kron(eye8, w1[1]), jnp.kron(eye8, root1)], axis=1)
    wt2p = jnp.pad(w2[1], ((0, 0), (0, HID - NCLS)))
    wmp = jnp.pad(root2, ((0, 0), (0, HID - NCLS)))
    w8t2 = jnp.kron(eye8, wt2p)
    w8m = jnp.kron(eye8, wmp)
    b1t = jnp.tile(b1, 8)[None, :]
    b2t = jnp.tile(jnp.pad(b2, (0, HID - NCLS)), 8)[None, :]
    m10 = (jnp.arange(128) % 16 == 10).astype(jnp.float32)[None, :]

    pad = EPAD - E
    src = jnp.concatenate([edge_index[0], jnp.zeros((pad,), jnp.int32)])
    dst = jnp.concatenate([edge_index[1],
                           jnp.full((pad,), N, jnp.int32)])
    src3 = src.reshape(NW, CPT, CHUNK)
    dst3 = dst.reshape(NW, CPT, CHUNK)

    t1p, xr1p = _tc_a(x8, w8a)
    pv1, pc1 = _sc_scatter_l1(src3, dst3, t1p.reshape(N, HID))
    t2p, miscp = _tc_b(pv1.reshape(NC, APK, 128), pc1.reshape(NC, APK, 128),
                       xr1p, w8t2, w8m, b1t, b2t, m10)
    pv2 = _sc_scatter_l2(src3, dst3, t2p.reshape(N, HID))
    res = _tc_c(pv2.reshape(NC, APK, 128), miscp)
    return res.reshape(N, HID)[:, :NCLS]


# speedup vs baseline: 34.5354x; 1.0871x over previous
import functools

import jax
import jax.numpy as jnp
from jax import lax
from jax.experimental import pallas as pl
from jax.experimental.pallas import tpu as pltpu
from jax.experimental.pallas import tpu_sc as plsc

N = 10000
E = 320000
F_IN = 128
HID = 16
NCLS = 10

NC = 2
NS = 16
NW = NC * NS
CHUNK = 128
CPT = 80
EPAD = NW * CPT * CHUNK
NACC = 10240
RPT = NACC // NS
NPK = N * HID // 128
APK = NACC * HID // 128


def _tc_a_body(x8_ref, w_ref, t_ref, xr_ref):
    y = jnp.dot(x8_ref[...], w_ref[...], preferred_element_type=jnp.float32)
    t_ref[...] = y[:, :128]
    xr_ref[...] = y[:, 128:]


def _tc_a(x8, w8a):
    return pl.pallas_call(
        _tc_a_body,
        grid=(1,),
        in_specs=[
            pl.BlockSpec((NPK, 8 * F_IN), lambda i: (0, 0)),
            pl.BlockSpec((8 * F_IN, 256), lambda i: (0, 0)),
        ],
        out_specs=[
            pl.BlockSpec((NPK, 128), lambda i: (0, 0)),
            pl.BlockSpec((NPK, 128), lambda i: (0, 0)),
        ],
        out_shape=[
            jax.ShapeDtypeStruct((NPK, 128), jnp.float32),
            jax.ShapeDtypeStruct((NPK, 128), jnp.float32),
        ],
    )(x8, w8a)


def _make_sc_scatter(with_counts):
    width = HID
    if with_counts:
        out_type = [
            jax.ShapeDtypeStruct((NC, NACC, width), jnp.float32),
            jax.ShapeDtypeStruct((NC, NACC, width), jnp.float32),
        ]
    else:
        out_type = jax.ShapeDtypeStruct((NC, NACC, width), jnp.float32)

    scratch = [
        pltpu.VMEM((CPT, CHUNK), jnp.int32),
        pltpu.VMEM((CPT, CHUNK), jnp.int32),
        [pltpu.VMEM((CHUNK, width), jnp.float32) for _ in range(16)],
        pltpu.VMEM((RPT, width), jnp.float32),
        pltpu.VMEM((CHUNK, width), jnp.float32),
        pltpu.VMEM_SHARED((N + 16, width), jnp.float32),
        pltpu.VMEM_SHARED((NACC, width), jnp.float32),
        pltpu.VMEM_SHARED((NACC, width), jnp.float32),
        pltpu.SemaphoreType.DMA,
        pltpu.SemaphoreType.DMA,
        pltpu.SemaphoreType.DMA,
        pltpu.SemaphoreType.DMA,
    ]

    @functools.partial(
        pl.kernel,
        out_type=out_type,
        mesh=plsc.VectorSubcoreMesh(core_axis_name="c", subcore_axis_name="s"),
        compiler_params=pltpu.CompilerParams(use_tc_tiling_on_sc=False,
                                             needs_layout_passes=False),
        scratch_types=scratch,
    )
    def sc_scatter(edges_hbm, table_hbm, *rest):
        if with_counts:
            out_hbm, cnt_hbm = rest[0], rest[1]
            scr = rest[2:]
        else:
            out_hbm = rest[0]
            scr = rest[1:]
        (src_v, dst_v, rbufs, zb_v, ones_v, table_spm, accum, accum_c,
         gsA, gsB, ssA, ssB) = scr
        bufA, bufB = rbufs[:8], rbufs[8:]

        cid = lax.axis_index("c")
        sid = lax.axis_index("s")
        wid = cid * NS + sid

        TS = N // NS
        icp1 = pltpu.async_copy(edges_hbm.at[wid], src_v, gsA)
        icp2 = pltpu.async_copy(edges_hbm.at[NW + wid], dst_v, gsB)
        icp3 = pltpu.async_copy(table_hbm.at[pl.ds(sid * TS, TS)],
                                table_spm.at[pl.ds(sid * TS, TS)], ssA)

        zeros16 = jnp.zeros((16,), jnp.float32)
        ones16 = jnp.ones((16,), jnp.float32)

        def zrow(i, carry):
            for u in range(8):
                zb_v[8 * i + u, pl.ds(0, 16)] = zeros16
            return carry

        lax.fori_loop(0, RPT // 8, zrow, 0)
        if with_counts:
            def orow(i, carry):
                for u in range(8):
                    ones_v[8 * i + u, pl.ds(0, 16)] = ones16
                return carry

            lax.fori_loop(0, CHUNK // 8, orow, 0)
        pltpu.sync_copy(zb_v, accum.at[pl.ds(sid * RPT, RPT)])
        if with_counts:
            pltpu.sync_copy(zb_v, accum_c.at[pl.ds(sid * RPT, RPT)])
        icp1.wait()
        icp2.wait()
        icp3.wait()
        plsc.subcore_barrier()

        def fire_gathers(c0, bufs, sem):
            for k in range(8):
                pltpu.async_copy(table_spm.at[src_v.at[c0 + k]], bufs[k], sem)

        def drain_gathers(bufs, sem):
            for k in range(8):
                pltpu.make_async_copy(
                    table_spm.at[src_v.at[0]], bufs[k], sem).wait()

        def fire_scatters(c0, bufs, sem):
            for k in range(8):
                pltpu.async_copy(bufs[k], accum.at[dst_v.at[c0 + k]], sem,
                                 add=True)
                if with_counts:
                    pltpu.async_copy(ones_v, accum_c.at[dst_v.at[c0 + k]],
                                     sem, add=True)

        def drain_scatters(bufs, sem):
            for k in range(8):
                pltpu.make_async_copy(
                    bufs[k], accum.at[dst_v.at[0]], sem).wait()
                if with_counts:
                    pltpu.make_async_copy(
                        ones_v, accum_c.at[dst_v.at[0]], sem).wait()

        NG = CPT // 16

        fire_gathers(0, bufA, gsA)

        def group_pair(i, carry):
            c0 = 16 * i

            @pl.when(i > 0)
            def _():
                drain_scatters(bufB, ssB)

            fire_gathers(c0 + 8, bufB, gsB)
            drain_gathers(bufA, gsA)
            fire_scatters(c0, bufA, ssA)

            drain_scatters(bufA, ssA)

            @pl.when(i < NG - 1)
            def _():
                fire_gathers(c0 + 16, bufA, gsA)

            drain_gathers(bufB, gsB)
            fire_scatters(c0 + 8, bufB, ssB)
            return carry

        lax.fori_loop(0, NG, group_pair, 0)
        drain_scatters(bufB, ssB)
        plsc.subcore_barrier()

        fcp1 = pltpu.async_copy(accum.at[pl.ds(sid * RPT, RPT)],
                                out_hbm.at[cid, pl.ds(sid * RPT, RPT)], gsA)
        if with_counts:
            pltpu.async_copy(accum_c.at[pl.ds(sid * RPT, RPT)],
                             cnt_hbm.at[cid, pl.ds(sid * RPT, RPT)],
                             gsB).wait()
        fcp1.wait()

    return sc_scatter


_sc_scatter_l1 = _make_sc_scatter(True)
_sc_scatter_l2 = _make_sc_scatter(False)


def _tc_b_body(pv_ref, pc_ref, xr_ref, wt2_ref, wm_ref, b1_ref, b2_ref,
               m10_ref, t2_ref, misc_ref):
    sp = pv_ref[0, :NPK] + pv_ref[1, :NPK]
    cntp = pc_ref[0, :NPK] + pc_ref[1, :NPK]
    invp = 1.0 / jnp.maximum(cntp, 1.0)
    pre = sp * invp + xr_ref[...] + b1_ref[...]
    h = jnp.where(pre > 0, pre,
                  jnp.exp(jnp.where(pre > 0, 0.0, pre)) - 1.0)
    t2_ref[...] = jnp.dot(h, wt2_ref[...], preferred_element_type=jnp.float32)
    hr2 = jnp.dot(h, wm_ref[...], preferred_element_type=jnp.float32)
    misc_ref[...] = hr2 + b2_ref[...] + invp * m10_ref[...]


def _tc_b(p1v, p1c, xr1p, w8t2, w8m, b1t, b2t, m10):
    vec = pl.BlockSpec((1, 128), lambda i: (0, 0))
    mat = pl.BlockSpec((128, 128), lambda i: (0, 0))
    pk = pl.BlockSpec((NPK, 128), lambda i: (0, 0))
    return pl.pallas_call(
        _tc_b_body,
        grid=(1,),
        in_specs=[
            pl.BlockSpec((NC, APK, 128), lambda i: (0, 0, 0)),
            pl.BlockSpec((NC, APK, 128), lambda i: (0, 0, 0)),
            pk, mat, mat, vec, vec, vec,
        ],
        out_specs=[pk, pk],
        out_shape=[
            jax.ShapeDtypeStruct((NPK, 128), jnp.float32),
            jax.ShapeDtypeStruct((NPK, 128), jnp.float32),
        ],
    )(p1v, p1c, xr1p, w8t2, w8m, b1t, b2t, m10)


def _tc_c_body(p_ref, misc_ref, out_ref):
    sp = p_ref[0, :NPK] + p_ref[1, :NPK]
    misc = misc_ref[...]

    lane = lax.broadcasted_iota(jnp.int32, (128, 128), 0)
    col = lax.broadcasted_iota(jnp.int32, (128, 128), 1)
    grp_eq = (lane // 16) == (col // 16)

    bmat = grp_eq & ((lane % 16) == 10) & ((col % 16) < NCLS)
    invb = jnp.dot(misc, bmat.astype(jnp.float32),
                   preferred_element_type=jnp.float32)

    lane1 = lax.broadcasted_iota(jnp.int32, (NPK, 128), 1)
    valid = (lane1 % 16) < NCLS
    o = jnp.where(valid, sp * invb + misc, -1e30)

    m = o
    for k in (1, 2, 4, 8):
        rot = grp_eq & ((col % 16) == ((lane + k) % 16))
        m = jnp.maximum(m, jnp.dot(m, rot.astype(jnp.float32),
                                   preferred_element_type=jnp.float32))
    z = o - m
    e = jnp.exp(z)
    ssum = jnp.dot(e, grp_eq.astype(jnp.float32),
                   preferred_element_type=jnp.float32)
    out_ref[...] = z - jnp.log(ssum)


def _tc_c(p2v, miscp):
    pk = pl.BlockSpec((NPK, 128), lambda i: (0, 0))
    return pl.pallas_call(
        _tc_c_body,
        grid=(1,),
        in_specs=[
            pl.BlockSpec((NC, APK, 128), lambda i: (0, 0, 0)),
            pk,
        ],
        out_specs=pk,
        out_shape=jax.ShapeDtypeStruct((NPK, 128), jnp.float32),
    )(p2v, miscp)


def kernel(x, edge_index, w1, root1, b1, w2, root2, b2):
    eye8 = jnp.eye(8, dtype=jnp.float32)
    x8 = x.reshape(NPK, 8 * F_IN)
    w8a = jnp.concatenate(
        [jnp.kron(eye8, w1[1]), jnp.kron(eye8, root1)], axis=1)
    wt2p = jnp.pad(w2[1], ((0, 0), (0, HID - NCLS)))
    wmp = jnp.pad(root2, ((0, 0), (0, HID - NCLS)))
    w8t2 = jnp.kron(eye8, wt2p)
    w8m = jnp.kron(eye8, wmp)
    b1t = jnp.tile(b1, 8)[None, :]
    b2t = jnp.tile(jnp.pad(b2, (0, HID - NCLS)), 8)[None, :]
    m10 = (jnp.arange(128) % 16 == 10).astype(jnp.float32)[None, :]

    e4 = jnp.pad(edge_index, ((0, 0), (0, EPAD - E)),
                 constant_values=N).reshape(2 * NW, CPT, CHUNK)

    t1p, xr1p = _tc_a(x8, w8a)
    pv1, pc1 = _sc_scatter_l1(e4, t1p.reshape(N, HID))
    t2p, miscp = _tc_b(pv1.reshape(NC, APK, 128), pc1.reshape(NC, APK, 128),
                       xr1p, w8t2, w8m, b1t, b2t, m10)
    pv2 = _sc_scatter_l2(e4, t2p.reshape(N, HID))
    res = _tc_c(pv2.reshape(NC, APK, 128), miscp)
    return res.reshape(N, HID)[:, :NCLS]
